# baseline (speedup 1.0000x reference)
import jax
import jax.numpy as jnp
from jax.experimental import pallas as pl
from jax.experimental.pallas import tpu as pltpu

_VOCAB = 1000000
_EMBED = 32
_NCLASS = 5
_DOCLEN = 200
_NBUF = 64


def _body(data_sm, tabT_hbm, w_ref, b_ref, out_ref, blk_ref, blk_ref2, sem, sem2):
    def _copy(t):
        v = data_sm[t]
        tc = pl.multiple_of((v // 128) * 128, 128)
        dst = blk_ref if t % 2 == 0 else blk_ref2
        sm = sem if t % 2 == 0 else sem2
        return pltpu.make_async_copy(
            tabT_hbm.at[:, pl.ds(tc, 128)],
            dst.at[pl.ds(_EMBED * (t // 2), _EMBED), :],
            sm.at[(t // 2) % _NBUF],
        )

    lane = jax.lax.broadcasted_iota(jnp.int32, (1, 128), 1)
    _NACC = 8
    accs = [jnp.zeros((_NCLASS, _EMBED), jnp.float32) for _ in range(_NACC)]

    def _process(t):
        vm = data_sm[t] % 128
        src = blk_ref if t % 2 == 0 else blk_ref2
        blk = src[_EMBED * (t // 2):_EMBED * (t // 2 + 1), :]
        mask = (lane == vm).astype(jnp.float32)
        col = jnp.sum(blk * mask, axis=1, keepdims=True)
        w_t = w_ref[:, pl.ds(t * _EMBED, _EMBED)]
        accs[t % _NACC] = accs[t % _NACC] + col.T * w_t

    _G = 16
    groups = [range(g, min(g + _G, _DOCLEN)) for g in range(0, _DOCLEN, _G)]

    def _start(gi):
        for t in groups[gi]:
            _copy(t).start()

    def _drain(gi):
        for t in groups[gi]:
            _copy(t).wait()
        for t in groups[gi]:
            _process(t)

    _start(0)
    _start(1)
    _start(2)
    for gi in range(3, len(groups)):
        _start(gi)
        _drain(gi - 3)
    _drain(len(groups) - 3)
    _drain(len(groups) - 2)
    _drain(len(groups) - 1)

    step = _NACC
    while step > 1:
        for i in range(step // 2):
            accs[i] = accs[i] + accs[i + step // 2]
        step //= 2
    logits = jnp.sum(accs[0], axis=1, keepdims=True).T + b_ref[...]
    m = jnp.max(logits, axis=1, keepdims=True)
    e = jnp.exp(logits - m)
    out_ref[...] = e / jnp.sum(e, axis=1, keepdims=True)


_tc_kernel = pl.pallas_call(
    _body,
    grid_spec=pltpu.PrefetchScalarGridSpec(
        num_scalar_prefetch=1,
        grid=(1,),
        in_specs=[
            pl.BlockSpec(memory_space=pl.ANY),
            pl.BlockSpec((_NCLASS, _EMBED * _DOCLEN), lambda i, *_: (0, 0)),
            pl.BlockSpec((1, _NCLASS), lambda i, *_: (0, 0)),
        ],
        out_specs=pl.BlockSpec((1, _NCLASS), lambda i, *_: (0, 0)),
        scratch_shapes=[
            pltpu.VMEM((_EMBED * _DOCLEN // 2, 128), jnp.float32),
            pltpu.VMEM((_EMBED * _DOCLEN // 2, 128), jnp.float32),
            pltpu.SemaphoreType.DMA((_NBUF,)),
            pltpu.SemaphoreType.DMA((_NBUF,)),
        ],
    ),
    out_shape=jax.ShapeDtypeStruct((1, _NCLASS), jnp.float32),
)


@jax.jit
def kernel(data, embed_table, W, b):
    data_i = data.astype(jnp.int32)
    tabT = embed_table.T
    return _tc_kernel(data_i, tabT, W, b.reshape(1, _NCLASS))


# speedup vs baseline: 1.0717x; 1.0717x over previous
import jax
import jax.numpy as jnp
from jax.experimental import pallas as pl
from jax.experimental.pallas import tpu as pltpu

_VOCAB = 1000000
_EMBED = 32
_NCLASS = 5
_DOCLEN = 200
_NBUF = 96


def _body(data_sm, tabT_hbm, w_ref, b_ref, out_ref, blk_ref, sem):
    def _copy(t):
        v = data_sm[t]
        tc = pl.multiple_of((v // 128) * 128, 128)
        return pltpu.make_async_copy(
            tabT_hbm.at[:, pl.ds(tc, 128)],
            blk_ref.at[pl.ds(_EMBED * t, _EMBED), :],
            sem.at[t % _NBUF],
        )

    lane = jax.lax.broadcasted_iota(jnp.int32, (1, 128), 1)
    _NACC = 8
    accs = [jnp.zeros((_NCLASS, _EMBED), jnp.float32) for _ in range(_NACC)]

    def _process(t):
        vm = data_sm[t] % 128
        blk = blk_ref[_EMBED * t:_EMBED * (t + 1), :]
        mask = (lane == vm).astype(jnp.float32)
        col = jnp.sum(blk * mask, axis=1, keepdims=True)
        w_t = w_ref[:, pl.ds(t * _EMBED, _EMBED)]
        accs[t % _NACC] = accs[t % _NACC] + col.T * w_t

    _G = 32
    groups = [range(g, min(g + _G, _DOCLEN)) for g in range(0, _DOCLEN, _G)]

    def _start(gi):
        for t in groups[gi]:
            _copy(t).start()

    def _drain(gi):
        for t in groups[gi]:
            _copy(t).wait()
        for t in groups[gi]:
            _process(t)

    _start(0)
    _start(1)
    _start(2)
    for gi in range(3, len(groups)):
        _start(gi)
        _drain(gi - 3)
    _drain(len(groups) - 3)
    _drain(len(groups) - 2)
    _drain(len(groups) - 1)

    step = _NACC
    while step > 1:
        for i in range(step // 2):
            accs[i] = accs[i] + accs[i + step // 2]
        step //= 2
    logits = jnp.sum(accs[0], axis=1, keepdims=True).T + b_ref[...]
    m = jnp.max(logits, axis=1, keepdims=True)
    e = jnp.exp(logits - m)
    out_ref[...] = e / jnp.sum(e, axis=1, keepdims=True)


_tc_kernel = pl.pallas_call(
    _body,
    grid_spec=pltpu.PrefetchScalarGridSpec(
        num_scalar_prefetch=1,
        grid=(1,),
        in_specs=[
            pl.BlockSpec(memory_space=pl.ANY),
            pl.BlockSpec((_NCLASS, _EMBED * _DOCLEN), lambda i, *_: (0, 0)),
            pl.BlockSpec((1, _NCLASS), lambda i, *_: (0, 0)),
        ],
        out_specs=pl.BlockSpec((1, _NCLASS), lambda i, *_: (0, 0)),
        scratch_shapes=[
            pltpu.VMEM((_EMBED * _DOCLEN, 128), jnp.float32),
            pltpu.SemaphoreType.DMA((_NBUF,)),
        ],
    ),
    out_shape=jax.ShapeDtypeStruct((1, _NCLASS), jnp.float32),
)


@jax.jit
def kernel(data, embed_table, W, b):
    data_i = data.astype(jnp.int32)
    tabT = embed_table.T
    return _tc_kernel(data_i, tabT, W, b.reshape(1, _NCLASS))
